# initial kernel scaffold (unmeasured)
import jax
import jax.numpy as jnp
from jax import lax
from jax.experimental import pallas as pl
from jax.experimental.pallas import tpu as pltpu

B, H, D, BS = 32, 16, 128, 32
NB = 256
LOCAL_PAGES = 256
T = 8
TOK = T * BS
STEPS = LOCAL_PAGES // T
SCALE = D ** -0.5
NEG = -1e30


def _compute_body(q_ref, k_ref, v_ref, bt_ref, lens_ref, m_ref, l_ref, acc_ref):
    step = pl.program_id(0)
    my_x = lax.axis_index("x")

    @pl.when(step == 0)
    def _init():
        m_ref[...] = jnp.full((H, B), NEG, jnp.float32)
        l_ref[...] = jnp.zeros((H, B), jnp.float32)
        acc_ref[...] = jnp.zeros((H, B, D), jnp.float32)

    q = q_ref[...]
    k = k_ref[...].reshape(TOK, H, D)
    v = v_ref[...].reshape(TOK, H, D)

    s = lax.dot_general(
        q, k, (((2,), (2,)), ((1,), (1,))),
        preferred_element_type=jnp.float32,
    ) * SCALE

    pid0 = step * T + my_x * LOCAL_PAGES
    pids = pid0 + lax.broadcasted_iota(jnp.int32, (1, T, 1), 1)
    hit = bt_ref[...][:, None, :] == pids
    valid = (
        lax.broadcasted_iota(jnp.int32, (1, 1, NB), 2)
        < lens_ref[...][:, None, :]
    )
    w = jnp.sum(jnp.where(hit & valid, 1.0, 0.0), axis=2)

    expand = (
        lax.broadcasted_iota(jnp.int32, (T, TOK), 1) // BS
        == lax.broadcasted_iota(jnp.int32, (T, TOK), 0)
    ).astype(jnp.float32)
    wt = lax.dot_general(
        w, expand, (((1,), (0,)), ((), ())),
        preferred_element_type=jnp.float32,
    )

    m_prev = m_ref[...]
    m_new = jnp.maximum(m_prev, jnp.max(s, axis=2))
    p = jnp.exp(s - m_new[:, :, None])
    pw = p * wt[None, :, :]
    alpha = jnp.exp(m_prev - m_new)
    l_ref[...] = l_ref[...] * alpha + jnp.sum(pw, axis=2)
    pv = lax.dot_general(
        pw, v, (((2,), (0,)), ((1,), (1,))),
        preferred_element_type=jnp.float32,
    )
    acc_ref[...] = acc_ref[...] * alpha[:, :, None] + pv
    m_ref[...] = m_new


def _compute(q, k, v, bt, lens2):
    return pl.pallas_call(
        _compute_body,
        grid=(STEPS,),
        in_specs=[
            pl.BlockSpec((B, H, D), lambda i: (0, 0, 0)),
            pl.BlockSpec((T, BS, H, D), lambda i: (i, 0, 0, 0)),
            pl.BlockSpec((T, BS, H, D), lambda i: (i, 0, 0, 0)),
            pl.BlockSpec((B, NB), lambda i: (0, 0)),
            pl.BlockSpec((B, 1), lambda i: (0, 0)),
        ],
        out_specs=[
            pl.BlockSpec((H, B), lambda i: (0, 0)),
            pl.BlockSpec((H, B), lambda i: (0, 0)),
            pl.BlockSpec((H, B, D), lambda i: (0, 0, 0)),
        ],
        out_shape=[
            jax.ShapeDtypeStruct((H, B), jnp.float32),
            jax.ShapeDtypeStruct((H, B), jnp.float32),
            jax.ShapeDtypeStruct((H, B, D), jnp.float32),
        ],
        compiler_params=pltpu.CompilerParams(
            dimension_semantics=("arbitrary",),
        ),
    )(q, k, v, bt, lens2)


def _exchange_body(
    m_ref, l_ref, acc_ref, out_ref, cm, cl, cacc, send_sems, recv_sems
):
    my_x = lax.axis_index("x")
    my_y = lax.axis_index("y")
    peer = (1 - my_x, my_y)

    barrier = pltpu.get_barrier_semaphore()
    pl.semaphore_signal(
        barrier, inc=1, device_id=peer, device_id_type=pl.DeviceIdType.MESH
    )
    pl.semaphore_wait(barrier, 1)

    copies = []
    for i, (src, dst) in enumerate(
        ((m_ref, cm), (l_ref, cl), (acc_ref, cacc))
    ):
        c = pltpu.make_async_remote_copy(
            src_ref=src,
            dst_ref=dst,
            send_sem=send_sems.at[i],
            recv_sem=recv_sems.at[i],
            device_id=peer,
            device_id_type=pl.DeviceIdType.MESH,
        )
        c.start()
        copies.append(c)
    for c in copies:
        c.wait()

    m0, l0, a0 = m_ref[...], l_ref[...], acc_ref[...]
    m1, l1, a1 = cm[...], cl[...], cacc[...]
    mm = jnp.maximum(m0, m1)
    e0 = jnp.exp(m0 - mm)
    e1 = jnp.exp(m1 - mm)
    ll = l0 * e0 + l1 * e1
    oo = (a0 * e0[:, :, None] + a1 * e1[:, :, None]) / ll[:, :, None]
    out_ref[...] = jnp.transpose(oo, (1, 0, 2))


def _exchange(m, l, acc):
    return pl.pallas_call(
        _exchange_body,
        out_shape=jax.ShapeDtypeStruct((B, H, D), jnp.float32),
        in_specs=[pl.BlockSpec(memory_space=pltpu.VMEM)] * 3,
        out_specs=pl.BlockSpec(memory_space=pltpu.VMEM),
        scratch_shapes=[
            pltpu.VMEM((H, B), jnp.float32),
            pltpu.VMEM((H, B), jnp.float32),
            pltpu.VMEM((H, B, D), jnp.float32),
            pltpu.SemaphoreType.DMA((3,)),
            pltpu.SemaphoreType.DMA((3,)),
        ],
        compiler_params=pltpu.CompilerParams(collective_id=0),
    )(m, l, acc)


def kernel(Q, K, V, bt, lens):
    q = Q.reshape(B, H, D)
    lens2 = lens.reshape(B, 1)
    m, l, acc = _compute(q, K, V, bt, lens2)
    out = _exchange(m, l, acc)
    return out.reshape(B, 1, H, D)


# baseline (device time: 337870 ns/iter reference)
import jax
import jax.numpy as jnp
from jax import lax
from jax.experimental import pallas as pl
from jax.experimental.pallas import tpu as pltpu

B, H, D, BS = 32, 16, 128, 32
NB = 256
LOCAL_PAGES = 256
T = 8
TOK = T * BS
STEPS = LOCAL_PAGES // T
SCALE = D ** -0.5
NEG = -1e30


def _compute_body(q_ref, k_ref, v_ref, bt_ref, lens_ref, m_ref, l_ref, acc_ref):
    step = pl.program_id(0)
    my_x = lax.axis_index("x")

    @pl.when(step == 0)
    def _init():
        m_ref[...] = jnp.full((H, B), NEG, jnp.float32)
        l_ref[...] = jnp.zeros((H, B), jnp.float32)
        acc_ref[...] = jnp.zeros((H, B, D), jnp.float32)

    q = q_ref[...]
    k = k_ref[...].reshape(TOK, H, D)
    v = v_ref[...].reshape(TOK, H, D)

    s = lax.dot_general(
        q, k, (((2,), (2,)), ((1,), (1,))),
        preferred_element_type=jnp.float32,
    ) * SCALE

    pid0 = step * T + my_x * LOCAL_PAGES
    pids = pid0 + lax.broadcasted_iota(jnp.int32, (1, T, 1), 1)
    hit = bt_ref[...][:, None, :] == pids
    valid = (
        lax.broadcasted_iota(jnp.int32, (1, 1, NB), 2)
        < lens_ref[...][:, None, :]
    )
    w = jnp.sum(jnp.where(hit & valid, 1.0, 0.0), axis=2)

    expand = (
        lax.broadcasted_iota(jnp.int32, (T, TOK), 1) // BS
        == lax.broadcasted_iota(jnp.int32, (T, TOK), 0)
    ).astype(jnp.float32)
    wt = lax.dot_general(
        w, expand, (((1,), (0,)), ((), ())),
        preferred_element_type=jnp.float32,
    )

    m_prev = m_ref[...]
    m_new = jnp.maximum(m_prev, jnp.max(s, axis=2))
    p = jnp.exp(s - m_new[:, :, None])
    pw = p * wt[None, :, :]
    alpha = jnp.exp(m_prev - m_new)
    l_ref[...] = l_ref[...] * alpha + jnp.sum(pw, axis=2)
    pv = lax.dot_general(
        pw, v, (((2,), (0,)), ((0,), (1,))),
        preferred_element_type=jnp.float32,
    )
    acc_ref[...] = acc_ref[...] * alpha[:, :, None] + pv
    m_ref[...] = m_new


def _compute(q, k, v, bt, lens2):
    return pl.pallas_call(
        _compute_body,
        grid=(STEPS,),
        in_specs=[
            pl.BlockSpec((B, H, D), lambda i: (0, 0, 0)),
            pl.BlockSpec((T, BS, H, D), lambda i: (i, 0, 0, 0)),
            pl.BlockSpec((T, BS, H, D), lambda i: (i, 0, 0, 0)),
            pl.BlockSpec((B, NB), lambda i: (0, 0)),
            pl.BlockSpec((B, 1), lambda i: (0, 0)),
        ],
        out_specs=[
            pl.BlockSpec((H, B), lambda i: (0, 0)),
            pl.BlockSpec((H, B), lambda i: (0, 0)),
            pl.BlockSpec((H, B, D), lambda i: (0, 0, 0)),
        ],
        out_shape=[
            jax.ShapeDtypeStruct((H, B), jnp.float32),
            jax.ShapeDtypeStruct((H, B), jnp.float32),
            jax.ShapeDtypeStruct((H, B, D), jnp.float32),
        ],
        compiler_params=pltpu.CompilerParams(
            dimension_semantics=("arbitrary",),
        ),
    )(q, k, v, bt, lens2)


def _exchange_body(
    m_ref, l_ref, acc_ref, out_ref, cm, cl, cacc, send_sems, recv_sems
):
    my_x = lax.axis_index("x")
    my_y = lax.axis_index("y")
    peer = (1 - my_x, my_y)

    barrier = pltpu.get_barrier_semaphore()
    pl.semaphore_signal(
        barrier, inc=1, device_id=peer, device_id_type=pl.DeviceIdType.MESH
    )
    pl.semaphore_wait(barrier, 1)

    copies = []
    for i, (src, dst) in enumerate(
        ((m_ref, cm), (l_ref, cl), (acc_ref, cacc))
    ):
        c = pltpu.make_async_remote_copy(
            src_ref=src,
            dst_ref=dst,
            send_sem=send_sems.at[i],
            recv_sem=recv_sems.at[i],
            device_id=peer,
            device_id_type=pl.DeviceIdType.MESH,
        )
        c.start()
        copies.append(c)
    for c in copies:
        c.wait()

    m0, l0, a0 = m_ref[...], l_ref[...], acc_ref[...]
    m1, l1, a1 = cm[...], cl[...], cacc[...]
    mm = jnp.maximum(m0, m1)
    e0 = jnp.exp(m0 - mm)
    e1 = jnp.exp(m1 - mm)
    ll = l0 * e0 + l1 * e1
    oo = (a0 * e0[:, :, None] + a1 * e1[:, :, None]) / ll[:, :, None]
    out_ref[...] = jnp.transpose(oo, (1, 0, 2))


def _exchange(m, l, acc):
    return pl.pallas_call(
        _exchange_body,
        out_shape=jax.ShapeDtypeStruct((B, H, D), jnp.float32),
        in_specs=[pl.BlockSpec(memory_space=pltpu.VMEM)] * 3,
        out_specs=pl.BlockSpec(memory_space=pltpu.VMEM),
        scratch_shapes=[
            pltpu.VMEM((H, B), jnp.float32),
            pltpu.VMEM((H, B), jnp.float32),
            pltpu.VMEM((H, B, D), jnp.float32),
            pltpu.SemaphoreType.DMA((3,)),
            pltpu.SemaphoreType.DMA((3,)),
        ],
        compiler_params=pltpu.CompilerParams(collective_id=0),
    )(m, l, acc)


def kernel(Q, K, V, bt, lens):
    q = Q.reshape(B, H, D)
    lens2 = lens.reshape(B, 1)
    m, l, acc = _compute(q, K, V, bt, lens2)
    out = _exchange(m, l, acc)
    return out.reshape(B, 1, H, D)


# device time: 177656 ns/iter; 1.9018x vs baseline; 1.9018x over previous
import jax
import jax.numpy as jnp
from jax import lax
from jax.experimental import pallas as pl
from jax.experimental.pallas import tpu as pltpu

B, H, D, BS = 32, 16, 128, 32
NB = 256
LP = 256
T = 64
TOKS = T * BS
NI = LP // T
SCALE = D ** -0.5


def _compute_body(q_ref, k_ref, v_ref, bt_ref, lens_ref, l_ref, acc_ref, wt_scr):
    h = pl.program_id(0)
    i = pl.program_id(1)
    my_x = lax.axis_index("x")

    @pl.when(h == 0)
    def _weights():
        pid0 = i * T + my_x * LP
        pids = pid0 + lax.broadcasted_iota(jnp.int32, (1, T, 1), 1)
        hit = bt_ref[...][:, None, :] == pids
        valid = (
            lax.broadcasted_iota(jnp.int32, (1, 1, NB), 2)
            < lens_ref[...][:, None, :]
        )
        w = jnp.sum(jnp.where(hit & valid, 1.0, 0.0), axis=2)
        expand = (
            lax.broadcasted_iota(jnp.int32, (T, TOKS), 1) // BS
            == lax.broadcasted_iota(jnp.int32, (T, TOKS), 0)
        ).astype(jnp.float32)
        wt_scr[i] = lax.dot_general(
            w, expand, (((1,), (0,)), ((), ())),
            preferred_element_type=jnp.float32,
        )

    @pl.when(i == 0)
    def _init():
        l_ref[...] = jnp.zeros((B, D), jnp.float32)
        acc_ref[...] = jnp.zeros((B, D), jnp.float32)

    s = lax.dot_general(
        q_ref[...], k_ref[...], (((1,), (1,)), ((), ())),
        preferred_element_type=jnp.float32,
    ) * SCALE
    p = jnp.exp(s) * wt_scr[i]
    l_ref[...] += jnp.broadcast_to(
        jnp.sum(p, axis=1, keepdims=True), (B, D)
    )
    acc_ref[...] += lax.dot_general(
        p, v_ref[...], (((1,), (0,)), ((), ())),
        preferred_element_type=jnp.float32,
    )


def _compute(q2, k2, v2, bt, lens2):
    return pl.pallas_call(
        _compute_body,
        grid=(H, NI),
        in_specs=[
            pl.BlockSpec((B, D), lambda h, i: (0, h)),
            pl.BlockSpec((TOKS, D), lambda h, i: (i, h)),
            pl.BlockSpec((TOKS, D), lambda h, i: (i, h)),
            pl.BlockSpec((B, NB), lambda h, i: (0, 0)),
            pl.BlockSpec((B, 1), lambda h, i: (0, 0)),
        ],
        out_specs=[
            pl.BlockSpec((B, D), lambda h, i: (0, h)),
            pl.BlockSpec((B, D), lambda h, i: (0, h)),
        ],
        out_shape=[
            jax.ShapeDtypeStruct((B, H * D), jnp.float32),
            jax.ShapeDtypeStruct((B, H * D), jnp.float32),
        ],
        scratch_shapes=[pltpu.VMEM((NI, B, TOKS), jnp.float32)],
        compiler_params=pltpu.CompilerParams(
            dimension_semantics=("arbitrary", "arbitrary"),
        ),
    )(q2, k2, v2, bt, lens2)


def _exchange_body(l_ref, acc_ref, out_ref, cl, cacc, send_sems, recv_sems):
    my_x = lax.axis_index("x")
    my_y = lax.axis_index("y")
    peer = (1 - my_x, my_y)

    barrier = pltpu.get_barrier_semaphore()
    pl.semaphore_signal(
        barrier, inc=1, device_id=peer, device_id_type=pl.DeviceIdType.MESH
    )
    pl.semaphore_wait(barrier, 1)

    copies = []
    for i, (src, dst) in enumerate(((l_ref, cl), (acc_ref, cacc))):
        c = pltpu.make_async_remote_copy(
            src_ref=src,
            dst_ref=dst,
            send_sem=send_sems.at[i],
            recv_sem=recv_sems.at[i],
            device_id=peer,
            device_id_type=pl.DeviceIdType.MESH,
        )
        c.start()
        copies.append(c)
    for c in copies:
        c.wait()

    ll = l_ref[...] + cl[...]
    aa = acc_ref[...] + cacc[...]
    out_ref[...] = aa / ll


def _exchange(l, acc):
    return pl.pallas_call(
        _exchange_body,
        out_shape=jax.ShapeDtypeStruct((B, H, D), jnp.float32),
        in_specs=[pl.BlockSpec(memory_space=pltpu.VMEM)] * 2,
        out_specs=pl.BlockSpec(memory_space=pltpu.VMEM),
        scratch_shapes=[
            pltpu.VMEM((B, H, D), jnp.float32),
            pltpu.VMEM((B, H, D), jnp.float32),
            pltpu.SemaphoreType.DMA((2,)),
            pltpu.SemaphoreType.DMA((2,)),
        ],
        compiler_params=pltpu.CompilerParams(collective_id=0),
    )(l, acc)


def kernel(Q, K, V, bt, lens):
    q2 = Q.reshape(B, H * D)
    k2 = K.reshape(LP * BS, H * D)
    v2 = V.reshape(LP * BS, H * D)
    lens2 = lens.reshape(B, 1)
    l, acc = _compute(q2, k2, v2, bt, lens2)
    out = _exchange(l.reshape(B, H, D), acc.reshape(B, H, D))
    return out.reshape(B, 1, H, D)


# device time: 84843 ns/iter; 3.9823x vs baseline; 2.0939x over previous
import jax
import jax.numpy as jnp
from jax import lax
from jax.experimental import pallas as pl
from jax.experimental.pallas import tpu as pltpu

B, H, D, BS = 32, 16, 128, 32
NB = 256
LP = 256
T = 64
TOKS = T * BS
NI = LP // T
NSTEPS = H * NI
SCALE = D ** -0.5


def _kv_copy(hbm_ref, buf, sems, step, slot, sem_idx):
    hh = step // NI
    ii = step % NI
    return pltpu.make_async_copy(
        hbm_ref.at[pl.ds(ii * T, T), :, hh, :],
        buf.at[slot],
        sems.at[sem_idx, slot],
    )


def _compute_body(
    q_ref, k_hbm, v_hbm, bt_ref, lens_ref, l_ref, acc_ref,
    kbuf, vbuf, wt_scr, sems,
):
    h = pl.program_id(0)
    i = pl.program_id(1)
    step = h * NI + i
    slot = step % 2
    my_x = lax.axis_index("x")

    @pl.when(step == 0)
    def _prologue():
        _kv_copy(k_hbm, kbuf, sems, step, slot, 0).start()
        _kv_copy(v_hbm, vbuf, sems, step, slot, 1).start()

    @pl.when(step + 1 < NSTEPS)
    def _prefetch():
        _kv_copy(k_hbm, kbuf, sems, step + 1, 1 - slot, 0).start()
        _kv_copy(v_hbm, vbuf, sems, step + 1, 1 - slot, 1).start()

    @pl.when(h == 0)
    def _weights():
        pid0 = i * T + my_x * LP
        pids = pid0 + lax.broadcasted_iota(jnp.int32, (1, T, 1), 1)
        hit = bt_ref[...][:, None, :] == pids
        valid = (
            lax.broadcasted_iota(jnp.int32, (1, 1, NB), 2)
            < lens_ref[...][:, None, :]
        )
        w = jnp.sum(jnp.where(hit & valid, 1.0, 0.0), axis=2)
        expand = (
            lax.broadcasted_iota(jnp.int32, (T, TOKS), 1) // BS
            == lax.broadcasted_iota(jnp.int32, (T, TOKS), 0)
        ).astype(jnp.float32)
        wt_scr[i] = lax.dot_general(
            w, expand, (((1,), (0,)), ((), ())),
            preferred_element_type=jnp.float32,
        )

    @pl.when(i == 0)
    def _init():
        l_ref[...] = jnp.zeros((B, D), jnp.float32)
        acc_ref[...] = jnp.zeros((B, D), jnp.float32)

    _kv_copy(k_hbm, kbuf, sems, step, slot, 0).wait()
    _kv_copy(v_hbm, vbuf, sems, step, slot, 1).wait()

    k = kbuf[slot].reshape(TOKS, D)
    v = vbuf[slot].reshape(TOKS, D)
    s = lax.dot_general(
        q_ref[...], k, (((1,), (1,)), ((), ())),
        preferred_element_type=jnp.float32,
    ) * SCALE
    p = jnp.exp(s) * wt_scr[i]
    l_ref[...] += jnp.broadcast_to(
        jnp.sum(p, axis=1, keepdims=True), (B, D)
    )
    acc_ref[...] += lax.dot_general(
        p, v, (((1,), (0,)), ((), ())),
        preferred_element_type=jnp.float32,
    )


def _compute(qt, k, v, bt, lens2):
    return pl.pallas_call(
        _compute_body,
        grid=(H, NI),
        in_specs=[
            pl.BlockSpec((None, B, D), lambda h, i: (h, 0, 0)),
            pl.BlockSpec(memory_space=pl.ANY),
            pl.BlockSpec(memory_space=pl.ANY),
            pl.BlockSpec((B, NB), lambda h, i: (0, 0)),
            pl.BlockSpec((B, 1), lambda h, i: (0, 0)),
        ],
        out_specs=[
            pl.BlockSpec((B, D), lambda h, i: (0, h)),
            pl.BlockSpec((B, D), lambda h, i: (0, h)),
        ],
        out_shape=[
            jax.ShapeDtypeStruct((B, H * D), jnp.float32),
            jax.ShapeDtypeStruct((B, H * D), jnp.float32),
        ],
        scratch_shapes=[
            pltpu.VMEM((2, T, BS, D), jnp.float32),
            pltpu.VMEM((2, T, BS, D), jnp.float32),
            pltpu.VMEM((NI, B, TOKS), jnp.float32),
            pltpu.SemaphoreType.DMA((2, 2)),
        ],
        compiler_params=pltpu.CompilerParams(
            dimension_semantics=("arbitrary", "arbitrary"),
        ),
    )(qt, k, v, bt, lens2)


def _exchange_body(l_ref, acc_ref, out_ref, cl, cacc, send_sems, recv_sems):
    my_x = lax.axis_index("x")
    my_y = lax.axis_index("y")
    peer = (1 - my_x, my_y)

    barrier = pltpu.get_barrier_semaphore()
    pl.semaphore_signal(
        barrier, inc=1, device_id=peer, device_id_type=pl.DeviceIdType.MESH
    )
    pl.semaphore_wait(barrier, 1)

    copies = []
    for i, (src, dst) in enumerate(((l_ref, cl), (acc_ref, cacc))):
        c = pltpu.make_async_remote_copy(
            src_ref=src,
            dst_ref=dst,
            send_sem=send_sems.at[i],
            recv_sem=recv_sems.at[i],
            device_id=peer,
            device_id_type=pl.DeviceIdType.MESH,
        )
        c.start()
        copies.append(c)
    for c in copies:
        c.wait()

    ll = l_ref[...] + cl[...]
    aa = acc_ref[...] + cacc[...]
    out_ref[...] = aa / ll


def _exchange(l, acc):
    return pl.pallas_call(
        _exchange_body,
        out_shape=jax.ShapeDtypeStruct((B, H * D), jnp.float32),
        in_specs=[pl.BlockSpec(memory_space=pltpu.VMEM)] * 2,
        out_specs=pl.BlockSpec(memory_space=pltpu.VMEM),
        scratch_shapes=[
            pltpu.VMEM((B, H * D), jnp.float32),
            pltpu.VMEM((B, H * D), jnp.float32),
            pltpu.SemaphoreType.DMA((2,)),
            pltpu.SemaphoreType.DMA((2,)),
        ],
        compiler_params=pltpu.CompilerParams(collective_id=0),
    )(l, acc)


def kernel(Q, K, V, bt, lens):
    qt = jnp.transpose(Q.reshape(B, H, D), (1, 0, 2))
    lens2 = lens.reshape(B, 1)
    l, acc = _compute(qt, K, V, bt, lens2)
    out = _exchange(l, acc)
    return out.reshape(B, 1, H, D)


# device time: 33494 ns/iter; 10.0875x vs baseline; 2.5331x over previous
import jax
import jax.numpy as jnp
from jax import lax
from jax.experimental import pallas as pl
from jax.experimental.pallas import tpu as pltpu

B, H, D, BS = 32, 16, 128, 32
NB = 256
LP = 256
TOKS = LP * BS
NC = H // 2
LAG = 2
NSTEPS = NC + LAG + 1
SCALE = D ** -0.5


def _kv_copy(hbm_ref, buf, sems, head, slot, sem_idx):
    return pltpu.make_async_copy(
        hbm_ref.at[:, :, head, :],
        buf.at[slot],
        sems.at[sem_idx, slot],
    )


def _x_copy(src, dst, send_sems, recv_sems, kind, j, peer):
    return pltpu.make_async_remote_copy(
        src_ref=src.at[j],
        dst_ref=dst.at[j],
        send_sem=send_sems.at[kind, j],
        recv_sem=recv_sems.at[kind, j],
        device_id=peer,
        device_id_type=pl.DeviceIdType.MESH,
    )


def _body(
    q_ref, k_hbm, v_hbm, bt_ref, lens_ref, out_hbm,
    kbuf, vbuf, wt_scr, own_l, own_acc, p_l, p_acc, res,
    dma_sems, x_send, x_recv, y_send, y_recv, out_sems,
):
    s_id = pl.program_id(0)
    j = jnp.minimum(s_id, NC - 1)
    slot = lax.rem(s_id, 2)
    my_x = lax.axis_index("x")
    my_y = lax.axis_index("y")
    x_peer = (1 - my_x, my_y)
    y_peer = (my_x, 1 - my_y)
    h0 = my_y * NC
    hg = h0 + j

    @pl.when(s_id == 0)
    def _prologue():
        _kv_copy(k_hbm, kbuf, dma_sems, h0, 0, 0).start()
        _kv_copy(v_hbm, vbuf, dma_sems, h0, 0, 1).start()

    @pl.when(s_id + 1 < NC)
    def _prefetch():
        _kv_copy(k_hbm, kbuf, dma_sems, hg + 1, 1 - slot, 0).start()
        _kv_copy(v_hbm, vbuf, dma_sems, hg + 1, 1 - slot, 1).start()

    @pl.when(s_id == 0)
    def _barrier():
        barrier = pltpu.get_barrier_semaphore()
        for peer in (x_peer, y_peer):
            pl.semaphore_signal(
                barrier, inc=1, device_id=peer,
                device_id_type=pl.DeviceIdType.MESH,
            )
        pl.semaphore_wait(barrier, 2)

    @pl.when(s_id == 0)
    def _weights():
        valid = (
            lax.broadcasted_iota(jnp.int32, (1, NB), 1) < lens_ref[...]
        )
        bt_m = jnp.where(valid, bt_ref[...], -1)
        pid0 = my_x * LP
        pids = pid0 + lax.broadcasted_iota(jnp.int32, (1, LP, 1), 1)
        hit = bt_m[:, None, :] == pids
        w = jnp.sum(hit.astype(jnp.float32), axis=2)
        wt_scr[0] = jnp.repeat(w, BS, axis=1)

    @pl.when(s_id < NC)
    def _compute():
        q = q_ref[:, 0, hg, :]
        _kv_copy(k_hbm, kbuf, dma_sems, hg, slot, 0).wait()
        k = kbuf[slot].reshape(TOKS, D)
        s = lax.dot_general(
            q, k, (((1,), (1,)), ((), ())),
            preferred_element_type=jnp.float32,
        ) * SCALE
        p = jnp.exp(s) * wt_scr[0]

        _kv_copy(v_hbm, vbuf, dma_sems, hg, slot, 1).wait()
        v = vbuf[slot].reshape(TOKS, D)
        own_l[j] = jnp.broadcast_to(
            jnp.sum(p, axis=1, keepdims=True), (B, D)
        )
        own_acc[j] = lax.dot_general(
            p, v, (((1,), (0,)), ((), ())),
            preferred_element_type=jnp.float32,
        )
        _x_copy(own_l, p_l, x_send, x_recv, 0, j, x_peer).start()
        _x_copy(own_acc, p_acc, x_send, x_recv, 1, j, x_peer).start()

    @pl.when((s_id >= LAG) & (s_id < NC + LAG))
    def _combine():
        jc = s_id - LAG
        cl = _x_copy(own_l, p_l, x_send, x_recv, 0, jc, x_peer)
        ca = _x_copy(own_acc, p_acc, x_send, x_recv, 1, jc, x_peer)
        cl.wait_send()
        ca.wait_send()
        cl.wait_recv()
        ca.wait_recv()
        res[h0 + jc] = (own_acc[jc] + p_acc[jc]) / (own_l[jc] + p_l[jc])

    @pl.when(s_id == NC + LAG - 1)
    def _y_exchange():
        pltpu.make_async_remote_copy(
            src_ref=res.at[pl.ds(h0, NC)],
            dst_ref=res.at[pl.ds(h0, NC)],
            send_sem=y_send.at[0],
            recv_sem=y_recv.at[0],
            device_id=y_peer,
            device_id_type=pl.DeviceIdType.MESH,
        ).start()

    @pl.when(s_id == NC + LAG)
    def _epilogue():
        yc = pltpu.make_async_remote_copy(
            src_ref=res.at[pl.ds(h0, NC)],
            dst_ref=res.at[pl.ds(h0, NC)],
            send_sem=y_send.at[0],
            recv_sem=y_recv.at[0],
            device_id=y_peer,
            device_id_type=pl.DeviceIdType.MESH,
        )
        yc.wait_send()
        yc.wait_recv()
        for hh in range(H):
            pltpu.make_async_copy(
                res.at[hh],
                out_hbm.at[:, 0, hh, :],
                out_sems.at[hh],
            ).start()
        for hh in range(H):
            pltpu.make_async_copy(
                res.at[hh],
                out_hbm.at[:, 0, hh, :],
                out_sems.at[hh],
            ).wait()


def _fused(q, k, v, bt, lens2):
    return pl.pallas_call(
        _body,
        grid=(NSTEPS,),
        in_specs=[
            pl.BlockSpec((B, 1, H, D), lambda s: (0, 0, 0, 0)),
            pl.BlockSpec(memory_space=pl.ANY),
            pl.BlockSpec(memory_space=pl.ANY),
            pl.BlockSpec((B, NB), lambda s: (0, 0)),
            pl.BlockSpec((B, 1), lambda s: (0, 0)),
        ],
        out_specs=pl.BlockSpec(memory_space=pl.ANY),
        out_shape=jax.ShapeDtypeStruct((B, 1, H, D), jnp.float32),
        scratch_shapes=[
            pltpu.VMEM((2, LP, BS, D), jnp.float32),
            pltpu.VMEM((2, LP, BS, D), jnp.float32),
            pltpu.VMEM((1, B, TOKS), jnp.float32),
            pltpu.VMEM((NC, B, D), jnp.float32),
            pltpu.VMEM((NC, B, D), jnp.float32),
            pltpu.VMEM((NC, B, D), jnp.float32),
            pltpu.VMEM((NC, B, D), jnp.float32),
            pltpu.VMEM((H, B, D), jnp.float32),
            pltpu.SemaphoreType.DMA((2, 2)),
            pltpu.SemaphoreType.DMA((2, NC)),
            pltpu.SemaphoreType.DMA((2, NC)),
            pltpu.SemaphoreType.DMA((1,)),
            pltpu.SemaphoreType.DMA((1,)),
            pltpu.SemaphoreType.DMA((H,)),
        ],
        compiler_params=pltpu.CompilerParams(
            dimension_semantics=("arbitrary",),
            collective_id=0,
        ),
    )(q, k, v, bt, lens2)


def kernel(Q, K, V, bt, lens):
    lens2 = lens.reshape(B, 1)
    return _fused(Q, K, V, bt, lens2)
